# baseline (speedup 1.0000x reference)
import jax
import jax.numpy as jnp
from jax import lax
from jax.experimental import pallas as pl
from jax.experimental.pallas import tpu as pltpu
from jax.experimental.pallas import tpu_sc as plsc

BATCH = 4096
HIST = 200
NDIM = 16
LANES = 16
NWORKERS = 32
ROWS_PER_W = BATCH // NWORKERS
BLK_ROWS = 4
NBLK = ROWS_PER_W // BLK_ROWS

_NCHUNKS = HIST // LANES + 1


def _start_block(i, ids_hbm, disc_hbm, diff_hbm, ids_v, rows_v, diff_v,
                 wrow0, p, sem_rows, sem_diff):
    row0 = wrow0 + i * BLK_ROWS
    pltpu.sync_copy(ids_hbm.at[pl.ds(row0, BLK_ROWS)], ids_v.at[p])
    for r in range(BLK_ROWS):
        pltpu.make_async_copy(disc_hbm.at[ids_v.at[p, r]], rows_v.at[p, r],
                              sem_rows).start()
        pltpu.make_async_copy(diff_hbm.at[ids_v.at[p, r]], diff_v.at[p, r],
                              sem_diff).start()


def _wait_block(ids_hbm, disc_hbm, diff_hbm, ids_v, rows_v, diff_v,
                p, sem_rows, sem_diff):
    for r in range(BLK_ROWS):
        pltpu.make_async_copy(disc_hbm.at[ids_v.at[p, r]], rows_v.at[p, r],
                              sem_rows).wait()
        pltpu.make_async_copy(diff_hbm.at[ids_v.at[p, r]], diff_v.at[p, r],
                              sem_diff).wait()


def _mirt_body(abil_hbm, ids_hbm, disc_hbm, diff_hbm, diff2d_hbm, out_hbm,
               ids_v, rows_v, diff_v, out_v, abil_v,
               sem_rows, sem_diff):
    nc = lax.axis_size("c")
    wid = lax.axis_index("s") * nc + lax.axis_index("c")
    wrow0 = wid * ROWS_PER_W

    pltpu.sync_copy(abil_hbm.at[pl.ds(wrow0, ROWS_PER_W)], abil_v)

    _start_block(0, ids_hbm, disc_hbm, diff_hbm, ids_v, rows_v, diff_v,
                 wrow0, 0, sem_rows, sem_diff)

    def block_body(i, carry):
        p = lax.rem(i, 2)
        _wait_block(ids_hbm, disc_hbm, diff_hbm, ids_v, rows_v, diff_v,
                    p, sem_rows, sem_diff)

        @pl.when(i + 1 < NBLK)
        def _():
            _start_block(i + 1, ids_hbm, disc_hbm, diff_hbm,
                         ids_v, rows_v, diff_v, wrow0, 1 - p,
                         sem_rows, sem_diff)

        lane = lax.iota(jnp.int32, LANES)

        def row_body(r, carry2):
            abil = abil_v[i * BLK_ROWS + r]

            def chunk_body(c, carry3):
                off = jnp.minimum(c * LANES, HIST - LANES)
                acc = jnp.zeros((LANES,), jnp.float32)
                for j in range(LANES):
                    w = rows_v[p, r, off + j]
                    s = plsc.cumsum(jnp.abs(w) * abil)[LANES - 1]
                    acc = jnp.where(lane == j, s, acc)
                x = acc - diff_v[p, r, pl.ds(off, LANES)]
                out_v[p, r, pl.ds(off, LANES)] = 1.0 / (1.0 + jnp.exp(-x))
                return carry3

            return lax.fori_loop(0, _NCHUNKS, chunk_body, carry2)

        lax.fori_loop(0, BLK_ROWS, row_body, 0)

        row0 = wrow0 + i * BLK_ROWS
        pltpu.sync_copy(out_v.at[p], out_hbm.at[pl.ds(row0, BLK_ROWS)])
        return carry

    lax.fori_loop(0, NBLK, block_body, 0)


def kernel(abilities, item_ids, disc_w, diff_w):
    ids32 = item_ids.astype(jnp.int32)
    diff_flat = diff_w.reshape(-1)

    mesh = plsc.VectorSubcoreMesh(core_axis_name="c", subcore_axis_name="s")

    run = pl.kernel(
        _mirt_body,
        out_type=jax.ShapeDtypeStruct((BATCH, HIST), jnp.float32),
        mesh=mesh,
        scratch_types=[
            pltpu.VMEM((2, BLK_ROWS, HIST), jnp.int32),
            pltpu.VMEM((2, BLK_ROWS, HIST, NDIM), jnp.float32),
            pltpu.VMEM((2, BLK_ROWS, HIST), jnp.float32),
            pltpu.VMEM((2, BLK_ROWS, HIST), jnp.float32),
            pltpu.VMEM((ROWS_PER_W, NDIM), jnp.float32),
            pltpu.SemaphoreType.DMA,
            pltpu.SemaphoreType.DMA,
        ],
        compiler_params=pltpu.CompilerParams(needs_layout_passes=False,
                                             use_tc_tiling_on_sc=False),
        name="neural_mirt_sc",
    )
    return run(abilities, ids32, disc_w, diff_flat, diff_w)


# speedup vs baseline: 2.4425x; 2.4425x over previous
import jax
import jax.numpy as jnp
from jax import lax
from jax.experimental import pallas as pl
from jax.experimental.pallas import tpu as pltpu
from jax.experimental.pallas import tpu_sc as plsc

BATCH = 4096
HIST = 200
NDIM = 16
LANES = 16
NWORKERS = 32
ROWS_PER_W = BATCH // NWORKERS
BLK_ROWS = 4
NBUF = 4
NBLK = ROWS_PER_W // BLK_ROWS

_NCHUNKS = HIST // LANES + 1


def _start_block(i, ids_hbm, disc_hbm, diff_hbm, ids_v, rows_v, diff_v,
                 wrow0, p, sem_rows, sem_diff):
    row0 = wrow0 + i * BLK_ROWS
    pltpu.sync_copy(ids_hbm.at[pl.ds(row0, BLK_ROWS)], ids_v.at[p])
    for r in range(BLK_ROWS):
        pltpu.make_async_copy(disc_hbm.at[ids_v.at[p, r]], rows_v.at[p, r],
                              sem_rows.at[p]).start()
        pltpu.make_async_copy(diff_hbm.at[ids_v.at[p, r]], diff_v.at[p, r],
                              sem_diff.at[p]).start()


def _wait_block(ids_hbm, disc_hbm, diff_hbm, ids_v, rows_v, diff_v,
                p, sem_rows, sem_diff):
    for r in range(BLK_ROWS):
        pltpu.make_async_copy(disc_hbm.at[ids_v.at[p, r]], rows_v.at[p, r],
                              sem_rows.at[p]).wait()
        pltpu.make_async_copy(diff_hbm.at[ids_v.at[p, r]], diff_v.at[p, r],
                              sem_diff.at[p]).wait()


def _mirt_body(abil_hbm, ids_hbm, disc_hbm, diff_hbm, out_hbm,
               ids_v, rows_v, diff_v, out_v, abil_v,
               sem_rows, sem_diff):
    nc = lax.axis_size("c")
    wid = lax.axis_index("s") * nc + lax.axis_index("c")
    wrow0 = wid * ROWS_PER_W

    pltpu.sync_copy(abil_hbm.at[pl.ds(wrow0, ROWS_PER_W)], abil_v)

    for b in range(NBUF - 1):
        _start_block(b, ids_hbm, disc_hbm, diff_hbm, ids_v, rows_v, diff_v,
                     wrow0, b, sem_rows, sem_diff)

    def block_body(i, carry):
        p = lax.rem(i, NBUF)
        _wait_block(ids_hbm, disc_hbm, diff_hbm, ids_v, rows_v, diff_v,
                    p, sem_rows, sem_diff)

        @pl.when(i + NBUF - 1 < NBLK)
        def _():
            _start_block(i + NBUF - 1, ids_hbm, disc_hbm, diff_hbm,
                         ids_v, rows_v, diff_v, wrow0,
                         lax.rem(i + NBUF - 1, NBUF),
                         sem_rows, sem_diff)

        lane = lax.iota(jnp.int32, LANES)

        def row_body(r, carry2):
            abil = abil_v[i * BLK_ROWS + r]

            def chunk_body(c, carry3):
                off = jnp.minimum(c * LANES, HIST - LANES)
                acc = jnp.zeros((LANES,), jnp.float32)
                for j in range(LANES):
                    w = rows_v[p, r, off + j]
                    s = plsc.cumsum(jnp.abs(w) * abil)[LANES - 1]
                    acc = jnp.where(lane == j, s, acc)
                x = acc - diff_v[p, r, pl.ds(off, LANES)]
                out_v[p, r, pl.ds(off, LANES)] = 1.0 / (1.0 + jnp.exp(-x))
                return carry3

            return lax.fori_loop(0, _NCHUNKS, chunk_body, carry2)

        lax.fori_loop(0, BLK_ROWS, row_body, 0)

        row0 = wrow0 + i * BLK_ROWS
        pltpu.sync_copy(out_v.at[p], out_hbm.at[pl.ds(row0, BLK_ROWS)])
        return carry

    lax.fori_loop(0, NBLK, block_body, 0)


def kernel(abilities, item_ids, disc_w, diff_w):
    ids32 = item_ids.astype(jnp.int32)
    diff_flat = diff_w.reshape(-1)

    mesh = plsc.VectorSubcoreMesh(core_axis_name="c", subcore_axis_name="s")

    run = pl.kernel(
        _mirt_body,
        out_type=jax.ShapeDtypeStruct((BATCH, HIST), jnp.float32),
        mesh=mesh,
        scratch_types=[
            pltpu.VMEM((NBUF, BLK_ROWS, HIST), jnp.int32),
            pltpu.VMEM((NBUF, BLK_ROWS, HIST, NDIM), jnp.float32),
            pltpu.VMEM((NBUF, BLK_ROWS, HIST), jnp.float32),
            pltpu.VMEM((NBUF, BLK_ROWS, HIST), jnp.float32),
            pltpu.VMEM((ROWS_PER_W, NDIM), jnp.float32),
            pltpu.SemaphoreType.DMA((NBUF,)),
            pltpu.SemaphoreType.DMA((NBUF,)),
        ],
        compiler_params=pltpu.CompilerParams(needs_layout_passes=False,
                                             use_tc_tiling_on_sc=False),
        name="neural_mirt_sc",
    )
    return run(abilities, ids32, disc_w, diff_flat)


# speedup vs baseline: 2.4467x; 1.0017x over previous
import jax
import jax.numpy as jnp
from jax import lax
from jax.experimental import pallas as pl
from jax.experimental.pallas import tpu as pltpu
from jax.experimental.pallas import tpu_sc as plsc

BATCH = 4096
HIST = 200
NDIM = 16
LANES = 16
NWORKERS = 32
ROWS_PER_W = BATCH // NWORKERS
BLK_ROWS = 4
NBUF = 2
NBLK = ROWS_PER_W // BLK_ROWS

_NCHUNKS = HIST // LANES + 1


def _start_block(i, ids_hbm, disc_hbm, diff_hbm, ids_v, rows_v, diff_v,
                 wrow0, p, sem_rows, sem_diff):
    row0 = wrow0 + i * BLK_ROWS
    pltpu.sync_copy(ids_hbm.at[pl.ds(row0, BLK_ROWS)], ids_v.at[p])
    for r in range(BLK_ROWS):
        pltpu.make_async_copy(disc_hbm.at[ids_v.at[p, r]], rows_v.at[p, r],
                              sem_rows.at[p]).start()
        pltpu.make_async_copy(diff_hbm.at[ids_v.at[p, r]], diff_v.at[p, r],
                              sem_diff.at[p]).start()


def _wait_block(ids_hbm, disc_hbm, diff_hbm, ids_v, rows_v, diff_v,
                p, sem_rows, sem_diff):
    for r in range(BLK_ROWS):
        pltpu.make_async_copy(disc_hbm.at[ids_v.at[p, r]], rows_v.at[p, r],
                              sem_rows.at[p]).wait()
        pltpu.make_async_copy(diff_hbm.at[ids_v.at[p, r]], diff_v.at[p, r],
                              sem_diff.at[p]).wait()


def _mirt_body(abil_hbm, ids_hbm, disc_hbm, diff_hbm, out_hbm,
               ids_v, rows_v, diff_v, out_v, abil_v,
               sem_rows, sem_diff):
    nc = lax.axis_size("c")
    wid = lax.axis_index("s") * nc + lax.axis_index("c")
    wrow0 = wid * ROWS_PER_W

    pltpu.sync_copy(abil_hbm.at[pl.ds(wrow0, ROWS_PER_W)], abil_v)

    for b in range(NBUF - 1):
        _start_block(b, ids_hbm, disc_hbm, diff_hbm, ids_v, rows_v, diff_v,
                     wrow0, b, sem_rows, sem_diff)

    def block_body(i, carry):
        p = lax.rem(i, NBUF)
        _wait_block(ids_hbm, disc_hbm, diff_hbm, ids_v, rows_v, diff_v,
                    p, sem_rows, sem_diff)

        @pl.when(i + NBUF - 1 < NBLK)
        def _():
            _start_block(i + NBUF - 1, ids_hbm, disc_hbm, diff_hbm,
                         ids_v, rows_v, diff_v, wrow0,
                         lax.rem(i + NBUF - 1, NBUF),
                         sem_rows, sem_diff)

        lane = lax.iota(jnp.int32, LANES)

        def row_body(r, carry2):
            abil = abil_v[i * BLK_ROWS + r]

            def chunk_body(c, carry3):
                off = jnp.minimum(c * LANES, HIST - LANES)
                acc = jnp.zeros((LANES,), jnp.float32)
                for j in range(LANES):
                    w = rows_v[p, r, off + j]
                    s = plsc.cumsum(jnp.abs(w) * abil)[LANES - 1]
                    acc = jnp.where(lane == j, s, acc)
                x = acc - diff_v[p, r, pl.ds(off, LANES)]
                out_v[p, r, pl.ds(off, LANES)] = 1.0 / (1.0 + jnp.exp(-x))
                return carry3

            return lax.fori_loop(0, _NCHUNKS, chunk_body, carry2)

        lax.fori_loop(0, BLK_ROWS, row_body, 0)

        row0 = wrow0 + i * BLK_ROWS
        pltpu.sync_copy(out_v.at[p], out_hbm.at[pl.ds(row0, BLK_ROWS)])
        return carry

    lax.fori_loop(0, NBLK, block_body, 0)


def kernel(abilities, item_ids, disc_w, diff_w):
    ids32 = item_ids.astype(jnp.int32)
    diff_flat = diff_w.reshape(-1)

    mesh = plsc.VectorSubcoreMesh(core_axis_name="c", subcore_axis_name="s")

    run = pl.kernel(
        _mirt_body,
        out_type=jax.ShapeDtypeStruct((BATCH, HIST), jnp.float32),
        mesh=mesh,
        scratch_types=[
            pltpu.VMEM((NBUF, BLK_ROWS, HIST), jnp.int32),
            pltpu.VMEM((NBUF, BLK_ROWS, HIST, NDIM), jnp.float32),
            pltpu.VMEM((NBUF, BLK_ROWS, HIST), jnp.float32),
            pltpu.VMEM((NBUF, BLK_ROWS, HIST), jnp.float32),
            pltpu.VMEM((ROWS_PER_W, NDIM), jnp.float32),
            pltpu.SemaphoreType.DMA((NBUF,)),
            pltpu.SemaphoreType.DMA((NBUF,)),
        ],
        compiler_params=pltpu.CompilerParams(needs_layout_passes=False,
                                             use_tc_tiling_on_sc=False),
        name="neural_mirt_sc",
    )
    return run(abilities, ids32, disc_w, diff_flat)
